# Initial kernel scaffold; baseline (speedup 1.0000x reference)
import jax
import jax.numpy as jnp
from jax.experimental import pallas as pl


def kernel(x, edge_index, edge_type, W, W_loop, b, gamma, beta):
    raise NotImplementedError("write your pallas kernel here")


# speedup vs baseline: 2.5938x; 2.5938x over previous
import functools

import jax
import jax.numpy as jnp
from jax import lax
from jax.experimental import pallas as pl
from jax.experimental.pallas import tpu as pltpu
from jax.experimental.pallas import tpu_sc as plsc


def _sc_geometry():
    try:
        info = plsc.get_sparse_core_info()
        return info.num_cores, info.num_subcores
    except Exception:
        return 2, 16


def _proj_body(pre_ref, scale_ref, shift_ref, wcat_ref, proj_ref, self_ref):
    hn = pre_ref[...] * scale_ref[...] + shift_ref[...]
    res = jnp.dot(hn, wcat_ref[...], preferred_element_type=jnp.float32)
    proj_ref[...] = res[:, :-self_ref.shape[1]]
    self_ref[...] = res[:, -self_ref.shape[1]:]


def _make_proj_call(N, D, R, BN):
    nb = N // BN
    return pl.pallas_call(
        _proj_body,
        grid=(nb,),
        in_specs=[
            pl.BlockSpec((BN, D), lambda n: (n, 0)),
            pl.BlockSpec((1, D), lambda n: (0, 0)),
            pl.BlockSpec((1, D), lambda n: (0, 0)),
            pl.BlockSpec((D, (R + 1) * D), lambda n: (0, 0)),
        ],
        out_specs=[
            pl.BlockSpec((BN, R * D), lambda n: (n, 0)),
            pl.BlockSpec((BN, D), lambda n: (n, 0)),
        ],
        out_shape=[
            jax.ShapeDtypeStruct((N, R * D), jnp.float32),
            jax.ShapeDtypeStruct((N, D), jnp.float32),
        ],
    )


def _make_sc_call(N, D, E):
    NC, NS = _sc_geometry()
    NW = NC * NS
    EPW = E // NW
    CH = 80
    NCHK = EPW // CH
    assert EPW * NW == E and NCHK * CH == EPW
    ZR = 40
    NCK = N // ZR
    assert NCK * ZR == N
    JMAX = (NCK + NS - 1) // NS

    mesh = plsc.VectorSubcoreMesh(core_axis_name="c", subcore_axis_name="s",
                                  num_cores=NC, num_subcores=NS)

    @functools.partial(
        pl.kernel,
        out_type=jax.ShapeDtypeStruct((NC * NCK, ZR, D), jnp.float32),
        mesh=mesh,
        scratch_types=[
            pltpu.VMEM((NCHK, CH), jnp.int32),
            pltpu.VMEM((NCHK, CH), jnp.int32),
            pltpu.VMEM((CH, D), jnp.float32),
            pltpu.VMEM((ZR, D), jnp.float32),
            pltpu.VMEM_SHARED((N, D), jnp.float32),
            pltpu.SemaphoreType.DMA,
        ],
    )
    def sc_fn(proj_hbm, gidx_hbm, dst_hbm, out_hbm,
              gix_v, dix_v, rows_v, stage_v, acc_sh, sem):
        c = lax.axis_index("c")
        s = lax.axis_index("s")
        wid = s * NC + c

        zeros16 = jnp.zeros((16,), jnp.float32)

        def zrow(r, _):
            for k in range(D // 16):
                stage_v[r, pl.ds(k * 16, 16)] = zeros16
            return 0

        lax.fori_loop(0, ZR, zrow, 0)
        for j in range(JMAX):
            ck = j * NS + s

            @pl.when(ck < NCK)
            def _():
                off = pl.multiple_of(ck * ZR, 8)
                pltpu.sync_copy(stage_v, acc_sh.at[pl.ds(off, ZR)])

        plsc.subcore_barrier()

        pltpu.sync_copy(gidx_hbm.at[wid], gix_v)
        pltpu.sync_copy(dst_hbm.at[wid], dix_v)

        def chunk(t, _):
            pltpu.async_copy(proj_hbm.at[gix_v.at[t]], rows_v, sem).wait()
            pltpu.sync_copy(rows_v, acc_sh.at[dix_v.at[t]], add=True)
            return 0

        lax.fori_loop(0, NCHK, chunk, 0)
        plsc.subcore_barrier()

        for j in range(JMAX):
            ck = j * NS + s

            @pl.when(ck < NCK)
            def _():
                off = pl.multiple_of(ck * ZR, 8)
                pltpu.sync_copy(acc_sh.at[pl.ds(off, ZR)], stage_v)
                pltpu.sync_copy(stage_v, out_hbm.at[c * NCK + ck])

    return sc_fn


def _combine_body(slabs_ref, selfp_ref, b_ref, gamma_ref, beta_ref,
                  pre_ref, stats_ref, sum_ref, sq_ref, *, nb, n_rows, relu):
    i = pl.program_id(0)
    outp = (slabs_ref[0] + slabs_ref[1] + selfp_ref[...] + b_ref[...])
    if relu:
        outp = jnp.maximum(outp, 0.0)
    pre_ref[...] = outp

    @pl.when(i == 0)
    def _():
        sum_ref[...] = jnp.zeros_like(sum_ref)
        sq_ref[...] = jnp.zeros_like(sq_ref)

    sum_ref[...] += jnp.sum(outp, axis=0, keepdims=True)
    sq_ref[...] += jnp.sum(outp * outp, axis=0, keepdims=True)

    @pl.when(i == nb - 1)
    def _():
        mean = sum_ref[...] / n_rows
        var = sq_ref[...] / n_rows - mean * mean
        scale = gamma_ref[...] * lax.rsqrt(var + 1e-5)
        shift = beta_ref[...] - mean * scale
        stats_ref[...] = jnp.concatenate([scale, shift], axis=0)


def _make_combine_call(N, D, BN, relu):
    nb = N // BN
    body = functools.partial(_combine_body, nb=nb, n_rows=float(N), relu=relu)
    return pl.pallas_call(
        body,
        grid=(nb,),
        in_specs=[
            pl.BlockSpec((2, BN, D), lambda n: (0, n, 0)),
            pl.BlockSpec((BN, D), lambda n: (n, 0)),
            pl.BlockSpec((1, D), lambda n: (0, 0)),
            pl.BlockSpec((1, D), lambda n: (0, 0)),
            pl.BlockSpec((1, D), lambda n: (0, 0)),
        ],
        out_specs=[
            pl.BlockSpec((BN, D), lambda n: (n, 0)),
            pl.BlockSpec((2, D), lambda n: (0, 0)),
        ],
        out_shape=[
            jax.ShapeDtypeStruct((N, D), jnp.float32),
            jax.ShapeDtypeStruct((2, D), jnp.float32),
        ],
        scratch_shapes=[
            pltpu.VMEM((1, D), jnp.float32),
            pltpu.VMEM((1, D), jnp.float32),
        ],
    )


def _final_body(pre_ref, stats_ref, out_ref):
    out_ref[...] = jnp.maximum(
        pre_ref[...] * stats_ref[0:1, :] + stats_ref[1:2, :], 0.0)


def _make_final_call(N, D, BN):
    nb = N // BN
    return pl.pallas_call(
        _final_body,
        grid=(nb,),
        in_specs=[
            pl.BlockSpec((BN, D), lambda n: (n, 0)),
            pl.BlockSpec((2, D), lambda n: (0, 0)),
        ],
        out_specs=pl.BlockSpec((BN, D), lambda n: (n, 0)),
        out_shape=jax.ShapeDtypeStruct((N, D), jnp.float32),
    )


def kernel(x, edge_index, edge_type, W, W_loop, b, gamma, beta):
    N, D = x.shape
    L, R, _, _ = W.shape
    E = edge_type.shape[0]
    BN = 1000

    NC, NS = _sc_geometry()
    NW = NC * NS
    EPW = E // NW
    CH = 80
    NCHK = EPW // CH

    src = edge_index[0]
    dst = edge_index[1]
    gidx3 = (src * R + edge_type).reshape(NW, NCHK, CH)
    dst3 = dst.reshape(NW, NCHK, CH)

    proj_call = _make_proj_call(N, D, R, BN)
    sc_call = _make_sc_call(N, D, E)
    comb_calls = [_make_combine_call(N, D, BN, relu=(l < L - 1))
                  for l in range(L)]
    final_call = _make_final_call(N, D, BN)

    scale = jnp.ones((1, D), jnp.float32)
    shift = jnp.zeros((1, D), jnp.float32)
    pre = x
    for l in range(L):
        w_cat = jnp.concatenate(
            [W[l].transpose(1, 0, 2).reshape(D, R * D), W_loop[l]], axis=1)
        proj, selfp = proj_call(pre, scale, shift, w_cat)
        slabs = sc_call(proj.reshape(N * R, D), gidx3, dst3).reshape(NC, N, D)
        pre, stats = comb_calls[l](
            slabs, selfp, b[l:l + 1], gamma[l:l + 1], beta[l:l + 1])
        scale = stats[0:1]
        shift = stats[1:2]
    return final_call(pre, stats)


# speedup vs baseline: 3.3022x; 1.2731x over previous
import functools

import jax
import jax.numpy as jnp
from jax import lax
from jax.experimental import pallas as pl
from jax.experimental.pallas import tpu as pltpu
from jax.experimental.pallas import tpu_sc as plsc


def _sc_geometry():
    try:
        info = plsc.get_sparse_core_info()
        return info.num_cores, info.num_subcores
    except Exception:
        return 2, 16


def _proj_body(pre_ref, scale_ref, shift_ref, wcat_ref, proj_ref, self_ref):
    hn = pre_ref[...] * scale_ref[...] + shift_ref[...]
    res = jnp.dot(hn, wcat_ref[...], preferred_element_type=jnp.float32)
    proj_ref[...] = res[:, :-self_ref.shape[1]]
    self_ref[...] = res[:, -self_ref.shape[1]:]


def _make_proj_call(N, D, R, BN):
    nb = N // BN
    return pl.pallas_call(
        _proj_body,
        grid=(nb,),
        in_specs=[
            pl.BlockSpec((BN, D), lambda n: (n, 0)),
            pl.BlockSpec((1, D), lambda n: (0, 0)),
            pl.BlockSpec((1, D), lambda n: (0, 0)),
            pl.BlockSpec((D, (R + 1) * D), lambda n: (0, 0)),
        ],
        out_specs=[
            pl.BlockSpec((BN, R * D), lambda n: (n, 0)),
            pl.BlockSpec((BN, D), lambda n: (n, 0)),
        ],
        out_shape=[
            jax.ShapeDtypeStruct((N, R * D), jnp.float32),
            jax.ShapeDtypeStruct((N, D), jnp.float32),
        ],
    )


def _make_sc_call(N, D, E):
    NC, NS = _sc_geometry()
    NW = NC * NS
    EPW = E // NW
    CH = 80
    NCHK = EPW // CH
    NG = 5
    GRP = NCHK // NG
    assert EPW * NW == E and NCHK * CH == EPW and GRP * NG == NCHK
    ZR = 40
    NCK = N // ZR
    assert NCK * ZR == N
    JMAX = (NCK + NS - 1) // NS

    mesh = plsc.VectorSubcoreMesh(core_axis_name="c", subcore_axis_name="s",
                                  num_cores=NC, num_subcores=NS)

    @functools.partial(
        pl.kernel,
        out_type=jax.ShapeDtypeStruct((NC * NCK, ZR, D), jnp.float32),
        mesh=mesh,
        scratch_types=[
            pltpu.VMEM((GRP, CH), jnp.int32),
            pltpu.VMEM((GRP, CH), jnp.int32),
            pltpu.VMEM((CH, D), jnp.float32),
            pltpu.VMEM((CH, D), jnp.float32),
            pltpu.VMEM((ZR, D), jnp.float32),
            pltpu.VMEM_SHARED((N, D), jnp.float32),
            pltpu.SemaphoreType.DMA,
            pltpu.SemaphoreType.DMA,
        ],
    )
    def sc_fn(proj_hbm, gidx_hbm, dst_hbm, out_hbm,
              gix_v, dix_v, rows_a, rows_b, stage_v, acc_sh, sem_a, sem_b):
        c = lax.axis_index("c")
        s = lax.axis_index("s")
        wid = s * NC + c

        zeros16 = jnp.zeros((16,), jnp.float32)

        def zrow(r, _):
            for k in range(D // 16):
                stage_v[r, pl.ds(k * 16, 16)] = zeros16
            return 0

        lax.fori_loop(0, ZR, zrow, 0)
        for j in range(JMAX):
            ck = j * NS + s

            @pl.when(ck < NCK)
            def _():
                off = pl.multiple_of(ck * ZR, 8)
                pltpu.sync_copy(stage_v, acc_sh.at[pl.ds(off, ZR)])

        plsc.subcore_barrier()

        def group(g, _):
            pltpu.sync_copy(gidx_hbm.at[wid, g], gix_v)
            pltpu.sync_copy(dst_hbm.at[wid, g], dix_v)
            pltpu.async_copy(proj_hbm.at[gix_v.at[0]], rows_a, sem_a)

            def pair(u, _):
                t0 = 2 * u
                t1 = t0 + 1

                @pl.when(t1 < GRP)
                def _():
                    pltpu.async_copy(proj_hbm.at[gix_v.at[t1]], rows_b, sem_b)

                pltpu.make_async_copy(
                    proj_hbm.at[gix_v.at[t0]], rows_a, sem_a).wait()
                pltpu.sync_copy(rows_a, acc_sh.at[dix_v.at[t0]], add=True)

                @pl.when(t0 + 2 < GRP)
                def _():
                    pltpu.async_copy(
                        proj_hbm.at[gix_v.at[t0 + 2]], rows_a, sem_a)

                @pl.when(t1 < GRP)
                def _():
                    pltpu.make_async_copy(
                        proj_hbm.at[gix_v.at[t1]], rows_b, sem_b).wait()
                    pltpu.sync_copy(rows_b, acc_sh.at[dix_v.at[t1]], add=True)

                return 0

            lax.fori_loop(0, (GRP + 1) // 2, pair, 0)
            return 0

        lax.fori_loop(0, NG, group, 0)
        plsc.subcore_barrier()

        for j in range(JMAX):
            ck = j * NS + s

            @pl.when(ck < NCK)
            def _():
                off = pl.multiple_of(ck * ZR, 8)
                pltpu.sync_copy(acc_sh.at[pl.ds(off, ZR)], stage_v)
                pltpu.sync_copy(stage_v, out_hbm.at[c * NCK + ck])

    return sc_fn


def _combine_body(slabs_ref, selfp_ref, b_ref, gamma_ref, beta_ref,
                  pre_ref, stats_ref, sum_ref, sq_ref, *, nb, n_rows, relu):
    i = pl.program_id(0)
    outp = (slabs_ref[0] + slabs_ref[1] + selfp_ref[...] + b_ref[...])
    if relu:
        outp = jnp.maximum(outp, 0.0)
    pre_ref[...] = outp

    @pl.when(i == 0)
    def _():
        sum_ref[...] = jnp.zeros_like(sum_ref)
        sq_ref[...] = jnp.zeros_like(sq_ref)

    sum_ref[...] += jnp.sum(outp, axis=0, keepdims=True)
    sq_ref[...] += jnp.sum(outp * outp, axis=0, keepdims=True)

    @pl.when(i == nb - 1)
    def _():
        mean = sum_ref[...] / n_rows
        var = sq_ref[...] / n_rows - mean * mean
        scale = gamma_ref[...] * lax.rsqrt(var + 1e-5)
        shift = beta_ref[...] - mean * scale
        stats_ref[...] = jnp.concatenate([scale, shift], axis=0)


def _make_combine_call(N, D, BN, relu):
    nb = N // BN
    body = functools.partial(_combine_body, nb=nb, n_rows=float(N), relu=relu)
    return pl.pallas_call(
        body,
        grid=(nb,),
        in_specs=[
            pl.BlockSpec((2, BN, D), lambda n: (0, n, 0)),
            pl.BlockSpec((BN, D), lambda n: (n, 0)),
            pl.BlockSpec((1, D), lambda n: (0, 0)),
            pl.BlockSpec((1, D), lambda n: (0, 0)),
            pl.BlockSpec((1, D), lambda n: (0, 0)),
        ],
        out_specs=[
            pl.BlockSpec((BN, D), lambda n: (n, 0)),
            pl.BlockSpec((2, D), lambda n: (0, 0)),
        ],
        out_shape=[
            jax.ShapeDtypeStruct((N, D), jnp.float32),
            jax.ShapeDtypeStruct((2, D), jnp.float32),
        ],
        scratch_shapes=[
            pltpu.VMEM((1, D), jnp.float32),
            pltpu.VMEM((1, D), jnp.float32),
        ],
    )


def _final_body(pre_ref, stats_ref, out_ref):
    out_ref[...] = jnp.maximum(
        pre_ref[...] * stats_ref[0:1, :] + stats_ref[1:2, :], 0.0)


def _make_final_call(N, D, BN):
    nb = N // BN
    return pl.pallas_call(
        _final_body,
        grid=(nb,),
        in_specs=[
            pl.BlockSpec((BN, D), lambda n: (n, 0)),
            pl.BlockSpec((2, D), lambda n: (0, 0)),
        ],
        out_specs=pl.BlockSpec((BN, D), lambda n: (n, 0)),
        out_shape=jax.ShapeDtypeStruct((N, D), jnp.float32),
    )


def kernel(x, edge_index, edge_type, W, W_loop, b, gamma, beta):
    N, D = x.shape
    L, R, _, _ = W.shape
    E = edge_type.shape[0]
    BN = 1000

    NC, NS = _sc_geometry()
    NW = NC * NS
    EPW = E // NW
    CH = 80
    NCHK = EPW // CH

    src = edge_index[0]
    dst = edge_index[1]
    NG = 5
    gidx4 = (src * R + edge_type).reshape(NW, NG, NCHK // NG, CH)
    dst4 = dst.reshape(NW, NG, NCHK // NG, CH)

    proj_call = _make_proj_call(N, D, R, BN)
    sc_call = _make_sc_call(N, D, E)
    comb_calls = [_make_combine_call(N, D, BN, relu=(l < L - 1))
                  for l in range(L)]
    final_call = _make_final_call(N, D, BN)

    scale = jnp.ones((1, D), jnp.float32)
    shift = jnp.zeros((1, D), jnp.float32)
    pre = x
    for l in range(L):
        w_cat = jnp.concatenate(
            [W[l].transpose(1, 0, 2).reshape(D, R * D), W_loop[l]], axis=1)
        proj, selfp = proj_call(pre, scale, shift, w_cat)
        slabs = sc_call(proj.reshape(N * R, D), gidx4, dst4).reshape(NC, N, D)
        pre, stats = comb_calls[l](
            slabs, selfp, b[l:l + 1], gamma[l:l + 1], beta[l:l + 1])
        scale = stats[0:1]
        shift = stats[1:2]
    return final_call(pre, stats)
